# Initial kernel scaffold; baseline (speedup 1.0000x reference)
import jax
import jax.numpy as jnp
from jax.experimental import pallas as pl


def kernel(features, edge_index, edge_types, edge_matrix, W_ih, W_hh, b_ih, b_hh):
    raise NotImplementedError("write your pallas kernel here")


# speedup vs baseline: 17.5506x; 17.5506x over previous
import functools

import jax
import jax.numpy as jnp
from jax import lax
from jax.experimental import pallas as pl
from jax.experimental.pallas import tpu as pltpu
from jax.experimental.pallas import tpu_sc as plsc

N = 10000
E = 160000
D = 32
T = 16
NC, NS, L = 2, 16, 16
NW = NC * NS
C = 128
K = 40
EPW = K * C
E_PAD = NW * EPW
NPAD = 10112
SLICE = NPAD // NS


def _tc_prep(features, B, srcm, typm):
    def body(f_ref, b_ref, s_ref, t_ref, p_ref, i_ref):
        p_ref[...] = jnp.dot(f_ref[...], b_ref[...],
                             preferred_element_type=jnp.float32)
        i_ref[...] = s_ref[...] * T + t_ref[...]

    return pl.pallas_call(
        body,
        grid=(10,),
        in_specs=[
            pl.BlockSpec((1000, D), lambda i: (i, 0)),
            pl.BlockSpec((D, T * D), lambda i: (0, 0)),
            pl.BlockSpec((128, C), lambda i: (i, 0)),
            pl.BlockSpec((128, C), lambda i: (i, 0)),
        ],
        out_specs=[
            pl.BlockSpec((1000, T * D), lambda i: (i, 0)),
            pl.BlockSpec((128, C), lambda i: (i, 0)),
        ],
        out_shape=[
            jax.ShapeDtypeStruct((N, T * D), jnp.float32),
            jax.ShapeDtypeStruct((E_PAD // C, C), jnp.int32),
        ],
    )(features, B, srcm, typm)


def _sc_agg(P_flat, idxw, dstw, zeros):
    mesh = plsc.VectorSubcoreMesh(core_axis_name="c", subcore_axis_name="s")

    @functools.partial(
        pl.kernel,
        out_type=jax.ShapeDtypeStruct((NC, NPAD, D), jnp.float32),
        mesh=mesh,
        compiler_params=pltpu.CompilerParams(use_tc_tiling_on_sc=False),
        scratch_types=[
            pltpu.VMEM((K, C), jnp.int32),
            pltpu.VMEM((K, C), jnp.int32),
            pltpu.VMEM((2, C, D), jnp.float32),
            pltpu.VMEM_SHARED((NPAD, D), jnp.float32),
            pltpu.SemaphoreType.DMA,
            pltpu.SemaphoreType.DMA,
        ],
    )
    def sc_kernel(p_hbm, idx_hbm, dst_hbm, z_hbm, out_hbm,
                  idx_v, dst_v, rows_v, agg, sem0, sem1):
        cid = lax.axis_index("c")
        sid = lax.axis_index("s")
        wid = sid * NC + cid
        pltpu.sync_copy(z_hbm.at[pl.ds(sid * SLICE, SLICE)],
                        agg.at[pl.ds(sid * SLICE, SLICE)])
        pltpu.sync_copy(idx_hbm.at[wid], idx_v)
        pltpu.sync_copy(dst_hbm.at[wid], dst_v)
        plsc.subcore_barrier()

        pltpu.async_copy(p_hbm.at[idx_v.at[0]], rows_v.at[0], sem0)

        def body(g, carry):
            j0 = 2 * g
            pltpu.async_copy(p_hbm.at[idx_v.at[j0 + 1]], rows_v.at[1], sem1)
            pltpu.make_async_copy(p_hbm.at[idx_v.at[j0]],
                                  rows_v.at[0], sem0).wait()
            pltpu.sync_copy(rows_v.at[0], agg.at[dst_v.at[j0]], add=True)

            @pl.when(g < K // 2 - 1)
            def _():
                pltpu.async_copy(p_hbm.at[idx_v.at[j0 + 2]], rows_v.at[0],
                                 sem0)

            pltpu.make_async_copy(p_hbm.at[idx_v.at[j0 + 1]],
                                  rows_v.at[1], sem1).wait()
            pltpu.sync_copy(rows_v.at[1], agg.at[dst_v.at[j0 + 1]], add=True)
            return carry

        lax.fori_loop(0, K // 2, body, 0)
        plsc.subcore_barrier()
        pltpu.sync_copy(agg.at[pl.ds(sid * SLICE, SLICE)],
                        out_hbm.at[cid, pl.ds(sid * SLICE, SLICE)])

    return sc_kernel(P_flat, idxw, dstw, zeros)


def _tc_gru(aggs, features, W_ih, W_hh, b_ih2, b_hh2):
    BN = 1000

    def body(a_ref, f_ref, wi_ref, wh_ref, bi_ref, bh_ref, o_ref):
        agg = a_ref[0] + a_ref[1]
        h = f_ref[...]
        gi = lax.dot_general(agg, wi_ref[...], (((1,), (1,)), ((), ())),
                             preferred_element_type=jnp.float32)
        gi = gi + bi_ref[0:1, :]
        gh = lax.dot_general(h, wh_ref[...], (((1,), (1,)), ((), ())),
                             preferred_element_type=jnp.float32)
        gh = gh + bh_ref[0:1, :]
        r = jax.nn.sigmoid(gi[:, :D] + gh[:, :D])
        z = jax.nn.sigmoid(gi[:, D:2 * D] + gh[:, D:2 * D])
        n = jnp.tanh(gi[:, 2 * D:] + r * gh[:, 2 * D:])
        o_ref[...] = (1.0 - z) * n + z * h

    return pl.pallas_call(
        body,
        grid=(N // BN,),
        in_specs=[
            pl.BlockSpec((NC, BN, D), lambda i: (0, i, 0)),
            pl.BlockSpec((BN, D), lambda i: (i, 0)),
            pl.BlockSpec((3 * D, D), lambda i: (0, 0)),
            pl.BlockSpec((3 * D, D), lambda i: (0, 0)),
            pl.BlockSpec((8, 3 * D), lambda i: (0, 0)),
            pl.BlockSpec((8, 3 * D), lambda i: (0, 0)),
        ],
        out_specs=pl.BlockSpec((BN, D), lambda i: (i, 0)),
        out_shape=jax.ShapeDtypeStruct((N, D), jnp.float32),
    )(aggs, features, W_ih, W_hh, b_ih2, b_hh2)


def kernel(features, edge_index, edge_types, edge_matrix, W_ih, W_hh, b_ih, b_hh):
    B = edge_matrix.reshape(T, D, D).transpose(2, 0, 1).reshape(D, T * D)
    src = edge_index[0]
    dst = edge_index[1]
    pad = E_PAD - E
    srcm = jnp.pad(src, (0, pad)).reshape(E_PAD // C, C)
    typm = jnp.pad(edge_types, (0, pad)).reshape(E_PAD // C, C)
    dstw = jnp.pad(dst, (0, pad), constant_values=N).reshape(NW, K, C)
    zeros = jnp.zeros((NPAD, D), jnp.float32)

    P, idx = _tc_prep(features, B, srcm, typm)
    aggs = _sc_agg(P.reshape(N * T, D), idx.reshape(NW, K, C), dstw, zeros)
    b_ih2 = jnp.broadcast_to(b_ih.reshape(1, 3 * D), (8, 3 * D))
    b_hh2 = jnp.broadcast_to(b_hh.reshape(1, 3 * D), (8, 3 * D))
    return _tc_gru(aggs, features, W_ih, W_hh, b_ih2, b_hh2)


# speedup vs baseline: 17.6517x; 1.0058x over previous
import functools

import jax
import jax.numpy as jnp
from jax import lax
from jax.experimental import pallas as pl
from jax.experimental.pallas import tpu as pltpu
from jax.experimental.pallas import tpu_sc as plsc

N = 10000
E = 160000
D = 32
T = 16
NC, NS, L = 2, 16, 16
NW = NC * NS
C = 128
K = 40
NBUF = 4
EPW = K * C
E_PAD = NW * EPW
NPAD = 10112
SLICE = NPAD // NS


def _tc_prep(features, B, b_ih2, b_hh2):
    def body(f_ref, b_ref, bi_ref, bh_ref, p_ref, bio_ref, bho_ref):
        p_ref[...] = jnp.dot(f_ref[...], b_ref[...],
                             preferred_element_type=jnp.float32)
        bio_ref[...] = bi_ref[...]
        bho_ref[...] = bh_ref[...]

    return pl.pallas_call(
        body,
        grid=(10,),
        in_specs=[
            pl.BlockSpec((1000, D), lambda i: (i, 0)),
            pl.BlockSpec((D, T * D), lambda i: (0, 0)),
            pl.BlockSpec((8, 3 * D), lambda i: (0, 0)),
            pl.BlockSpec((8, 3 * D), lambda i: (0, 0)),
        ],
        out_specs=[
            pl.BlockSpec((1000, T * D), lambda i: (i, 0)),
            pl.BlockSpec((8, 3 * D), lambda i: (0, 0)),
            pl.BlockSpec((8, 3 * D), lambda i: (0, 0)),
        ],
        out_shape=[
            jax.ShapeDtypeStruct((N, T * D), jnp.float32),
            jax.ShapeDtypeStruct((8, 3 * D), jnp.float32),
            jax.ShapeDtypeStruct((8, 3 * D), jnp.float32),
        ],
    )(features, B, b_ih2, b_hh2)


def _sc_agg(P_flat, srcw, typw, dstw, zeros):
    mesh = plsc.VectorSubcoreMesh(core_axis_name="c", subcore_axis_name="s")

    @functools.partial(
        pl.kernel,
        out_type=jax.ShapeDtypeStruct((NC, NPAD, D), jnp.float32),
        mesh=mesh,
        compiler_params=pltpu.CompilerParams(use_tc_tiling_on_sc=False),
        scratch_types=[
            pltpu.VMEM((K, C), jnp.int32),
            pltpu.VMEM((K, C), jnp.int32),
            pltpu.VMEM((K, C), jnp.int32),
            pltpu.VMEM((K, C), jnp.int32),
            pltpu.VMEM((NBUF, C, D), jnp.float32),
            pltpu.VMEM_SHARED((NPAD, D), jnp.float32),
            [pltpu.SemaphoreType.DMA] * NBUF,
        ],
    )
    def sc_kernel(p_hbm, src_hbm, typ_hbm, dst_hbm, z_hbm, out_hbm,
                  src_v, typ_v, idx_v, dst_v, rows_v, agg, sems):
        cid = lax.axis_index("c")
        sid = lax.axis_index("s")
        wid = sid * NC + cid
        pltpu.sync_copy(z_hbm, agg.at[pl.ds(sid * SLICE, SLICE)])
        pltpu.sync_copy(src_hbm.at[wid], src_v)
        pltpu.sync_copy(typ_hbm.at[wid], typ_v)
        pltpu.sync_copy(dst_hbm.at[wid], dst_v)
        for j in range(K):
            for i in range(C // L):
                s = src_v[j, pl.ds(i * L, L)]
                t = typ_v[j, pl.ds(i * L, L)]
                idx_v[j, pl.ds(i * L, L)] = s * T + t
        plsc.subcore_barrier()

        for b in range(NBUF - 1):
            pltpu.async_copy(p_hbm.at[idx_v.at[b]], rows_v.at[b], sems[b])

        def round_body(g, carry):
            for b in range(NBUF):
                j = NBUF * g + b
                nb = (b + NBUF - 1) % NBUF

                @pl.when(j + NBUF - 1 < K)
                def _():
                    pltpu.async_copy(p_hbm.at[idx_v.at[j + NBUF - 1]],
                                     rows_v.at[nb], sems[nb])

                pltpu.make_async_copy(p_hbm.at[idx_v.at[j]],
                                      rows_v.at[b], sems[b]).wait()
                pltpu.sync_copy(rows_v.at[b], agg.at[dst_v.at[j]], add=True)
            return carry

        lax.fori_loop(0, K // NBUF, round_body, 0)
        plsc.subcore_barrier()
        pltpu.sync_copy(agg.at[pl.ds(sid * SLICE, SLICE)],
                        out_hbm.at[cid, pl.ds(sid * SLICE, SLICE)])

    return sc_kernel(P_flat, srcw, typw, dstw, zeros)


def _tc_gru(aggs, features, W_ih, W_hh, b_ih2, b_hh2):
    BN = 1000

    def body(a_ref, f_ref, wi_ref, wh_ref, bi_ref, bh_ref, o_ref):
        agg = a_ref[0] + a_ref[1]
        h = f_ref[...]
        gi = lax.dot_general(agg, wi_ref[...], (((1,), (1,)), ((), ())),
                             preferred_element_type=jnp.float32)
        gi = gi + bi_ref[0:1, :]
        gh = lax.dot_general(h, wh_ref[...], (((1,), (1,)), ((), ())),
                             preferred_element_type=jnp.float32)
        gh = gh + bh_ref[0:1, :]
        r = jax.nn.sigmoid(gi[:, :D] + gh[:, :D])
        z = jax.nn.sigmoid(gi[:, D:2 * D] + gh[:, D:2 * D])
        n = jnp.tanh(gi[:, 2 * D:] + r * gh[:, 2 * D:])
        o_ref[...] = (1.0 - z) * n + z * h

    return pl.pallas_call(
        body,
        grid=(N // BN,),
        in_specs=[
            pl.BlockSpec((NC, BN, D), lambda i: (0, i, 0)),
            pl.BlockSpec((BN, D), lambda i: (i, 0)),
            pl.BlockSpec((3 * D, D), lambda i: (0, 0)),
            pl.BlockSpec((3 * D, D), lambda i: (0, 0)),
            pl.BlockSpec((8, 3 * D), lambda i: (0, 0)),
            pl.BlockSpec((8, 3 * D), lambda i: (0, 0)),
        ],
        out_specs=pl.BlockSpec((BN, D), lambda i: (i, 0)),
        out_shape=jax.ShapeDtypeStruct((N, D), jnp.float32),
    )(aggs, features, W_ih, W_hh, b_ih2, b_hh2)


def kernel(features, edge_index, edge_types, edge_matrix, W_ih, W_hh, b_ih, b_hh):
    B = edge_matrix.reshape(T, D, D).transpose(2, 0, 1).reshape(D, T * D)
    src = edge_index[0]
    dst = edge_index[1]
    pad = E_PAD - E
    srcw = jnp.pad(src, (0, pad)).reshape(NW, K, C)
    typw = jnp.pad(edge_types, (0, pad)).reshape(NW, K, C)
    dstw = jnp.pad(dst, (0, pad), constant_values=N).reshape(NW, K, C)
    zeros = jnp.zeros((SLICE, D), jnp.float32)

    b_ih2 = jnp.broadcast_to(b_ih.reshape(1, 3 * D), (8, 3 * D))
    b_hh2 = jnp.broadcast_to(b_hh.reshape(1, 3 * D), (8, 3 * D))
    P, b_ih3, b_hh3 = _tc_prep(features, B, b_ih2, b_hh2)
    aggs = _sc_agg(P.reshape(N * T, D), srcw, typw, dstw, zeros)
    return _tc_gru(aggs, features, W_ih, W_hh, b_ih3, b_hh3)


# speedup vs baseline: 17.6787x; 1.0015x over previous
import functools

import jax
import jax.numpy as jnp
from jax import lax
from jax.experimental import pallas as pl
from jax.experimental.pallas import tpu as pltpu
from jax.experimental.pallas import tpu_sc as plsc

N = 10000
E = 160000
D = 32
T = 16
NC, NS, L = 2, 16, 16
NW = NC * NS
C = 128
K = 40
NBUF = 8
EPW = K * C
E_PAD = NW * EPW
NPAD = 10112
SLICE = NPAD // NS


def _tc_prep(features, B, b_ih2, b_hh2):
    def body(f_ref, b_ref, bi_ref, bh_ref, p_ref, bio_ref, bho_ref):
        p_ref[...] = jnp.dot(f_ref[...], b_ref[...],
                             preferred_element_type=jnp.float32)
        bio_ref[...] = bi_ref[...]
        bho_ref[...] = bh_ref[...]

    return pl.pallas_call(
        body,
        grid=(10,),
        in_specs=[
            pl.BlockSpec((1000, D), lambda i: (i, 0)),
            pl.BlockSpec((D, T * D), lambda i: (0, 0)),
            pl.BlockSpec((8, 3 * D), lambda i: (0, 0)),
            pl.BlockSpec((8, 3 * D), lambda i: (0, 0)),
        ],
        out_specs=[
            pl.BlockSpec((1000, T * D), lambda i: (i, 0)),
            pl.BlockSpec((8, 3 * D), lambda i: (0, 0)),
            pl.BlockSpec((8, 3 * D), lambda i: (0, 0)),
        ],
        out_shape=[
            jax.ShapeDtypeStruct((N, T * D), jnp.float32),
            jax.ShapeDtypeStruct((8, 3 * D), jnp.float32),
            jax.ShapeDtypeStruct((8, 3 * D), jnp.float32),
        ],
    )(features, B, b_ih2, b_hh2)


def _sc_agg(P_flat, srcw, typw, dstw, zeros):
    mesh = plsc.VectorSubcoreMesh(core_axis_name="c", subcore_axis_name="s")

    @functools.partial(
        pl.kernel,
        out_type=jax.ShapeDtypeStruct((NC, NPAD, D), jnp.float32),
        mesh=mesh,
        compiler_params=pltpu.CompilerParams(use_tc_tiling_on_sc=False),
        scratch_types=[
            pltpu.VMEM((K, C), jnp.int32),
            pltpu.VMEM((K, C), jnp.int32),
            pltpu.VMEM((K, C), jnp.int32),
            pltpu.VMEM((K, C), jnp.int32),
            pltpu.VMEM((NBUF, C, D), jnp.float32),
            pltpu.VMEM_SHARED((NPAD, D), jnp.float32),
            [pltpu.SemaphoreType.DMA] * NBUF,
        ],
    )
    def sc_kernel(p_hbm, src_hbm, typ_hbm, dst_hbm, z_hbm, out_hbm,
                  src_v, typ_v, idx_v, dst_v, rows_v, agg, sems):
        cid = lax.axis_index("c")
        sid = lax.axis_index("s")
        wid = sid * NC + cid
        pltpu.sync_copy(z_hbm, agg.at[pl.ds(sid * SLICE, SLICE)])
        pltpu.sync_copy(src_hbm.at[wid], src_v)
        pltpu.sync_copy(typ_hbm.at[wid], typ_v)
        pltpu.sync_copy(dst_hbm.at[wid], dst_v)
        for j in range(K):
            for i in range(C // L):
                s = src_v[j, pl.ds(i * L, L)]
                t = typ_v[j, pl.ds(i * L, L)]
                idx_v[j, pl.ds(i * L, L)] = s * T + t
        plsc.subcore_barrier()

        for b in range(NBUF - 1):
            pltpu.async_copy(p_hbm.at[idx_v.at[b]], rows_v.at[b], sems[b])

        def round_body(g, carry):
            for b in range(NBUF):
                j = NBUF * g + b
                nb = (b + NBUF - 1) % NBUF

                @pl.when(j + NBUF - 1 < K)
                def _():
                    pltpu.async_copy(p_hbm.at[idx_v.at[j + NBUF - 1]],
                                     rows_v.at[nb], sems[nb])

                pltpu.make_async_copy(p_hbm.at[idx_v.at[j]],
                                      rows_v.at[b], sems[b]).wait()
                pltpu.sync_copy(rows_v.at[b], agg.at[dst_v.at[j]], add=True)
            return carry

        lax.fori_loop(0, K // NBUF, round_body, 0)
        plsc.subcore_barrier()
        pltpu.sync_copy(agg.at[pl.ds(sid * SLICE, SLICE)],
                        out_hbm.at[cid, pl.ds(sid * SLICE, SLICE)])

    return sc_kernel(P_flat, srcw, typw, dstw, zeros)


def _tc_gru(aggs, features, W_ih, W_hh, b_ih2, b_hh2):
    BN = 1000

    def body(a_ref, f_ref, wi_ref, wh_ref, bi_ref, bh_ref, o_ref):
        agg = a_ref[0] + a_ref[1]
        h = f_ref[...]
        gi = lax.dot_general(agg, wi_ref[...], (((1,), (1,)), ((), ())),
                             preferred_element_type=jnp.float32)
        gi = gi + bi_ref[0:1, :]
        gh = lax.dot_general(h, wh_ref[...], (((1,), (1,)), ((), ())),
                             preferred_element_type=jnp.float32)
        gh = gh + bh_ref[0:1, :]
        r = jax.nn.sigmoid(gi[:, :D] + gh[:, :D])
        z = jax.nn.sigmoid(gi[:, D:2 * D] + gh[:, D:2 * D])
        n = jnp.tanh(gi[:, 2 * D:] + r * gh[:, 2 * D:])
        o_ref[...] = (1.0 - z) * n + z * h

    return pl.pallas_call(
        body,
        grid=(N // BN,),
        in_specs=[
            pl.BlockSpec((NC, BN, D), lambda i: (0, i, 0)),
            pl.BlockSpec((BN, D), lambda i: (i, 0)),
            pl.BlockSpec((3 * D, D), lambda i: (0, 0)),
            pl.BlockSpec((3 * D, D), lambda i: (0, 0)),
            pl.BlockSpec((8, 3 * D), lambda i: (0, 0)),
            pl.BlockSpec((8, 3 * D), lambda i: (0, 0)),
        ],
        out_specs=pl.BlockSpec((BN, D), lambda i: (i, 0)),
        out_shape=jax.ShapeDtypeStruct((N, D), jnp.float32),
    )(aggs, features, W_ih, W_hh, b_ih2, b_hh2)


def kernel(features, edge_index, edge_types, edge_matrix, W_ih, W_hh, b_ih, b_hh):
    B = edge_matrix.reshape(T, D, D).transpose(2, 0, 1).reshape(D, T * D)
    src = edge_index[0]
    dst = edge_index[1]
    pad = E_PAD - E
    srcw = jnp.pad(src, (0, pad)).reshape(NW, K, C)
    typw = jnp.pad(edge_types, (0, pad)).reshape(NW, K, C)
    dstw = jnp.pad(dst, (0, pad), constant_values=N).reshape(NW, K, C)
    zeros = jnp.zeros((SLICE, D), jnp.float32)

    b_ih2 = jnp.broadcast_to(b_ih.reshape(1, 3 * D), (8, 3 * D))
    b_hh2 = jnp.broadcast_to(b_hh.reshape(1, 3 * D), (8, 3 * D))
    P, b_ih3, b_hh3 = _tc_prep(features, B, b_ih2, b_hh2)
    aggs = _sc_agg(P.reshape(N * T, D), srcw, typw, dstw, zeros)
    return _tc_gru(aggs, features, W_ih, W_hh, b_ih3, b_hh3)
